# Initial kernel scaffold; baseline (speedup 1.0000x reference)
import jax
import jax.numpy as jnp
from jax.experimental import pallas as pl


def kernel(text, emb_table, W1, b1, W2, b2):
    raise NotImplementedError("write your pallas kernel here")


# speedup vs baseline: 2.2089x; 2.2089x over previous
import functools

import jax
import jax.numpy as jnp
from jax import lax
from jax.experimental import pallas as pl
from jax.experimental.pallas import tpu as pltpu
from jax.experimental.pallas import tpu_sc as plsc

VOCAB = 1000000
EMBED_DIM = 64
HIDDEN = 128
NUM_CLASS = 10
BATCH = 16384
SEQ = 50

NC, NS = 2, 16
NW = NC * NS
BAGS_PER_W = BATCH // NW
IDX_PER_W = BAGS_PER_W * SEQ
GROUP = 128
GROUPS_PER_W = IDX_PER_W // GROUP


def _emb_bag_sum(text_g, table, bagids, zeros):
    mesh = plsc.VectorSubcoreMesh(core_axis_name="c", subcore_axis_name="s")

    @functools.partial(
        pl.kernel,
        mesh=mesh,
        compiler_params=pltpu.CompilerParams(use_tc_tiling_on_sc=False),
        out_type=jax.ShapeDtypeStruct((BATCH, EMBED_DIM), jnp.float32),
        scratch_types=[
            pltpu.VMEM((GROUPS_PER_W, GROUP), jnp.int32),
            pltpu.VMEM((GROUPS_PER_W, GROUP), jnp.int32),
            pltpu.VMEM((GROUP, EMBED_DIM), jnp.float32),
            pltpu.VMEM_SHARED((NS * BAGS_PER_W, EMBED_DIM), jnp.float32),
        ],
    )
    def k(text_hbm, table_hbm, bagid_hbm, zeros_hbm, out_hbm,
          idx_v, bagid_v, rows_v, acc_sh):
        cid = lax.axis_index("c")
        sid = lax.axis_index("s")
        wid = sid * NC + cid
        pltpu.sync_copy(text_hbm.at[pl.ds(wid * GROUPS_PER_W, GROUPS_PER_W)],
                        idx_v)
        pltpu.sync_copy(bagid_hbm.at[sid], bagid_v)
        pltpu.sync_copy(zeros_hbm, acc_sh.at[pl.ds(sid * BAGS_PER_W,
                                                   BAGS_PER_W)])

        @pl.loop(0, GROUPS_PER_W)
        def _(g):
            pltpu.sync_copy(table_hbm.at[idx_v.at[g]], rows_v)
            pltpu.sync_copy(rows_v, acc_sh.at[bagid_v.at[g]], add=True)

        pltpu.sync_copy(acc_sh.at[pl.ds(sid * BAGS_PER_W, BAGS_PER_W)],
                        out_hbm.at[pl.ds(wid * BAGS_PER_W, BAGS_PER_W)])

    return k(text_g, table, bagids, zeros)


_BM = 1024


def _mlp_body(x_ref, w1_ref, b1_ref, w2_ref, b2_ref, o_ref):
    x = jnp.dot(x_ref[...], w1_ref[...], preferred_element_type=jnp.float32)
    x = jnp.maximum(x + b1_ref[...], 0.0)
    o_ref[...] = (
        jnp.dot(x, w2_ref[...], preferred_element_type=jnp.float32)
        + b2_ref[...])


def _mlp(x, w1, b1, w2, b2):
    return pl.pallas_call(
        _mlp_body,
        grid=(BATCH // _BM,),
        in_specs=[
            pl.BlockSpec((_BM, EMBED_DIM), lambda i: (i, 0)),
            pl.BlockSpec((EMBED_DIM, HIDDEN), lambda i: (0, 0)),
            pl.BlockSpec((1, HIDDEN), lambda i: (0, 0)),
            pl.BlockSpec((HIDDEN, NUM_CLASS), lambda i: (0, 0)),
            pl.BlockSpec((1, NUM_CLASS), lambda i: (0, 0)),
        ],
        out_specs=pl.BlockSpec((_BM, NUM_CLASS), lambda i: (i, 0)),
        out_shape=jax.ShapeDtypeStruct((BATCH, NUM_CLASS), jnp.float32),
    )(x, w1, b1, w2, b2)


def kernel(text, emb_table, W1, b1, W2, b2):
    text_g = text.reshape(NW * GROUPS_PER_W, GROUP)
    local = (jnp.arange(IDX_PER_W, dtype=jnp.int32) // SEQ).reshape(
        1, GROUPS_PER_W, GROUP)
    offs = (jnp.arange(NS, dtype=jnp.int32) * BAGS_PER_W).reshape(NS, 1, 1)
    bagids = local + offs
    zeros = jnp.zeros((BAGS_PER_W, EMBED_DIM), jnp.float32)
    sums = _emb_bag_sum(text_g, emb_table, bagids, zeros)
    w1s = W1 * (1.0 / SEQ)
    return _mlp(sums, w1s, b1.reshape(1, HIDDEN), W2, b2.reshape(1, NUM_CLASS))


# speedup vs baseline: 2.6047x; 1.1792x over previous
import functools

import jax
import jax.numpy as jnp
from jax import lax
from jax.experimental import pallas as pl
from jax.experimental.pallas import tpu as pltpu
from jax.experimental.pallas import tpu_sc as plsc

VOCAB = 1000000
EMBED_DIM = 64
HIDDEN = 128
NUM_CLASS = 10
BATCH = 16384
SEQ = 50

NC, NS = 2, 16
NW = NC * NS
BAGS_PER_W = BATCH // NW
IDX_PER_W = BAGS_PER_W * SEQ
GROUP = 128
GROUPS_PER_W = IDX_PER_W // GROUP
NBUF = 5


def _emb_bag_sum(text_g, table, bagids, zeros):
    mesh = plsc.VectorSubcoreMesh(core_axis_name="c", subcore_axis_name="s")

    @functools.partial(
        pl.kernel,
        mesh=mesh,
        compiler_params=pltpu.CompilerParams(use_tc_tiling_on_sc=False),
        out_type=jax.ShapeDtypeStruct((BATCH, EMBED_DIM), jnp.float32),
        scratch_types=[
            pltpu.VMEM((GROUPS_PER_W, GROUP), jnp.int32),
            pltpu.VMEM((GROUPS_PER_W, GROUP), jnp.int32),
        ] + [pltpu.VMEM((GROUP, EMBED_DIM), jnp.float32)] * NBUF + [
            pltpu.VMEM_SHARED((NS * BAGS_PER_W, EMBED_DIM), jnp.float32),
        ] + [pltpu.SemaphoreType.DMA] * NBUF,
    )
    def k(text_hbm, table_hbm, bagid_hbm, zeros_hbm, out_hbm,
          idx_v, bagid_v, *rest):
        rows_bufs = rest[:NBUF]
        acc_sh = rest[NBUF]
        sems = rest[NBUF + 1:]
        cid = lax.axis_index("c")
        sid = lax.axis_index("s")
        wid = sid * NC + cid
        pltpu.sync_copy(text_hbm.at[pl.ds(wid * GROUPS_PER_W, GROUPS_PER_W)],
                        idx_v)
        pltpu.sync_copy(bagid_hbm.at[sid], bagid_v)
        pltpu.sync_copy(zeros_hbm, acc_sh.at[pl.ds(sid * BAGS_PER_W,
                                                   BAGS_PER_W)])

        bufs = tuple(zip(rows_bufs, sems))
        for p in range(NBUF - 1):
            pltpu.async_copy(table_hbm.at[idx_v.at[p]], bufs[p][0],
                             bufs[p][1])

        @pl.loop(0, GROUPS_PER_W, step=NBUF)
        def _(g):
            for j in range(NBUF):
                rows, sem = bufs[j]
                nrows, nsem = bufs[(j + NBUF - 1) % NBUF]
                pltpu.make_async_copy(table_hbm.at[idx_v.at[g + j]],
                                      rows, sem).wait()

                @pl.when(g + j + NBUF - 1 < GROUPS_PER_W)
                def _():
                    pltpu.async_copy(table_hbm.at[idx_v.at[g + j + NBUF - 1]],
                                     nrows, nsem)

                pltpu.sync_copy(rows, acc_sh.at[bagid_v.at[g + j]], add=True)

        pltpu.sync_copy(acc_sh.at[pl.ds(sid * BAGS_PER_W, BAGS_PER_W)],
                        out_hbm.at[pl.ds(wid * BAGS_PER_W, BAGS_PER_W)])

    return k(text_g, table, bagids, zeros)


_BM = 1024


def _mlp_body(x_ref, w1_ref, b1_ref, w2_ref, b2_ref, o_ref):
    x = jnp.dot(x_ref[...], w1_ref[...], preferred_element_type=jnp.float32)
    x = jnp.maximum(x + b1_ref[...], 0.0)
    o_ref[...] = (
        jnp.dot(x, w2_ref[...], preferred_element_type=jnp.float32)
        + b2_ref[...])


def _mlp(x, w1, b1, w2, b2):
    return pl.pallas_call(
        _mlp_body,
        grid=(BATCH // _BM,),
        in_specs=[
            pl.BlockSpec((_BM, EMBED_DIM), lambda i: (i, 0)),
            pl.BlockSpec((EMBED_DIM, HIDDEN), lambda i: (0, 0)),
            pl.BlockSpec((1, HIDDEN), lambda i: (0, 0)),
            pl.BlockSpec((HIDDEN, NUM_CLASS), lambda i: (0, 0)),
            pl.BlockSpec((1, NUM_CLASS), lambda i: (0, 0)),
        ],
        out_specs=pl.BlockSpec((_BM, NUM_CLASS), lambda i: (i, 0)),
        out_shape=jax.ShapeDtypeStruct((BATCH, NUM_CLASS), jnp.float32),
    )(x, w1, b1, w2, b2)


def kernel(text, emb_table, W1, b1, W2, b2):
    text_g = text.reshape(NW * GROUPS_PER_W, GROUP)
    local = (jnp.arange(IDX_PER_W, dtype=jnp.int32) // SEQ).reshape(
        1, GROUPS_PER_W, GROUP)
    offs = (jnp.arange(NS, dtype=jnp.int32) * BAGS_PER_W).reshape(NS, 1, 1)
    bagids = local + offs
    zeros = jnp.zeros((BAGS_PER_W, EMBED_DIM), jnp.float32)
    sums = _emb_bag_sum(text_g, emb_table, bagids, zeros)
    w1s = W1 * (1.0 / SEQ)
    return _mlp(sums, w1s, b1.reshape(1, HIDDEN), W2, b2.reshape(1, NUM_CLASS))


# speedup vs baseline: 2.7200x; 1.0443x over previous
import functools

import jax
import jax.numpy as jnp
from jax import lax
from jax.experimental import pallas as pl
from jax.experimental.pallas import tpu as pltpu
from jax.experimental.pallas import tpu_sc as plsc

VOCAB = 1000000
EMBED_DIM = 64
HIDDEN = 128
NUM_CLASS = 10
BATCH = 16384
SEQ = 50

NC, NS = 2, 16
NW = NC * NS
BAGS_PER_W = BATCH // NW
IDX_PER_W = BAGS_PER_W * SEQ
GROUP = 128
GROUPS_PER_W = IDX_PER_W // GROUP
CHUNKS = BAGS_PER_W // GROUP
NBUF = 8


def _emb_bag_sum(text_t, table, bagids, zeros):
    mesh = plsc.VectorSubcoreMesh(core_axis_name="c", subcore_axis_name="s")

    @functools.partial(
        pl.kernel,
        mesh=mesh,
        compiler_params=pltpu.CompilerParams(use_tc_tiling_on_sc=False),
        out_type=jax.ShapeDtypeStruct((BATCH, EMBED_DIM), jnp.float32),
        scratch_types=[
            pltpu.VMEM((SEQ, BAGS_PER_W), jnp.int32),
            pltpu.VMEM((CHUNKS, GROUP), jnp.int32),
        ] + [pltpu.VMEM((GROUP, EMBED_DIM), jnp.float32)] * NBUF + [
            pltpu.VMEM_SHARED((NS * BAGS_PER_W, EMBED_DIM), jnp.float32),
        ] + [pltpu.SemaphoreType.DMA] * NBUF,
    )
    def k(text_hbm, table_hbm, bagid_hbm, zeros_hbm, out_hbm,
          idx_v, bagid_v, *rest):
        rows_bufs = rest[:NBUF]
        acc_sh = rest[NBUF]
        sems = rest[NBUF + 1:]
        cid = lax.axis_index("c")
        sid = lax.axis_index("s")
        wid = sid * NC + cid
        pltpu.sync_copy(text_hbm.at[:, pl.ds(wid * BAGS_PER_W, BAGS_PER_W)],
                        idx_v)
        pltpu.sync_copy(bagid_hbm.at[sid], bagid_v)
        pltpu.sync_copy(zeros_hbm, acc_sh.at[pl.ds(sid * BAGS_PER_W,
                                                   BAGS_PER_W)])

        bufs = tuple(zip(rows_bufs, sems))

        def idx_slice(t):
            ci = t // SEQ
            s = t - ci * SEQ
            return idx_v.at[s, pl.ds(ci * GROUP, GROUP)]

        def bag_slice(t):
            return bagid_v.at[t // SEQ]

        for p in range(NBUF - 1):
            pltpu.async_copy(table_hbm.at[idx_slice(p)], bufs[p][0],
                             bufs[p][1])

        @pl.loop(0, GROUPS_PER_W, step=NBUF)
        def _(g):
            for j in range(NBUF):
                rows, sem = bufs[j]
                nrows, nsem = bufs[(j + NBUF - 1) % NBUF]
                pltpu.make_async_copy(table_hbm.at[idx_slice(g + j)],
                                      rows, sem).wait()

                @pl.when(g + j + NBUF - 1 < GROUPS_PER_W)
                def _():
                    pltpu.async_copy(table_hbm.at[idx_slice(g + j + NBUF - 1)],
                                     nrows, nsem)

                pltpu.sync_copy(rows, acc_sh.at[bag_slice(g + j)], add=True)

        pltpu.sync_copy(acc_sh.at[pl.ds(sid * BAGS_PER_W, BAGS_PER_W)],
                        out_hbm.at[pl.ds(wid * BAGS_PER_W, BAGS_PER_W)])

    return k(text_t, table, bagids, zeros)


_BM = 1024


def _mlp_body(x_ref, w1_ref, b1_ref, w2_ref, b2_ref, o_ref):
    x = jnp.dot(x_ref[...], w1_ref[...], preferred_element_type=jnp.float32)
    x = jnp.maximum(x + b1_ref[...], 0.0)
    o_ref[...] = (
        jnp.dot(x, w2_ref[...], preferred_element_type=jnp.float32)
        + b2_ref[...])


def _mlp(x, w1, b1, w2, b2):
    return pl.pallas_call(
        _mlp_body,
        grid=(BATCH // _BM,),
        in_specs=[
            pl.BlockSpec((_BM, EMBED_DIM), lambda i: (i, 0)),
            pl.BlockSpec((EMBED_DIM, HIDDEN), lambda i: (0, 0)),
            pl.BlockSpec((1, HIDDEN), lambda i: (0, 0)),
            pl.BlockSpec((HIDDEN, NUM_CLASS), lambda i: (0, 0)),
            pl.BlockSpec((1, NUM_CLASS), lambda i: (0, 0)),
        ],
        out_specs=pl.BlockSpec((_BM, NUM_CLASS), lambda i: (i, 0)),
        out_shape=jax.ShapeDtypeStruct((BATCH, NUM_CLASS), jnp.float32),
    )(x, w1, b1, w2, b2)


def kernel(text, emb_table, W1, b1, W2, b2):
    local = (jnp.arange(BAGS_PER_W, dtype=jnp.int32)).reshape(
        1, CHUNKS, GROUP)
    offs = (jnp.arange(NS, dtype=jnp.int32) * BAGS_PER_W).reshape(NS, 1, 1)
    bagids = local + offs
    zeros = jnp.zeros((BAGS_PER_W, EMBED_DIM), jnp.float32)
    sums = _emb_bag_sum(text.T, emb_table, bagids, zeros)
    w1s = W1 * (1.0 / SEQ)
    return _mlp(sums, w1s, b1.reshape(1, HIDDEN), W2, b2.reshape(1, NUM_CLASS))
